# baseline (speedup 1.0000x reference)
import functools

import jax
import jax.numpy as jnp
from jax import lax
from jax.experimental import pallas as pl
from jax.experimental.pallas import tpu as pltpu
from jax.experimental.pallas import tpu_sc as plsc

N = 10000
NP = 10240
B = 2
E = 160000
EP = 163840
EMB = 128
HEMB = EMB // 2
CHUNK = 128
NCHUNK = EP // CHUNK
TILES = 16
CPT = NCHUNK // TILES
STRIPE = NP // TILES
SUBBLK = STRIPE // CHUNK

_MESH = plsc.VectorSubcoreMesh(core_axis_name="c", subcore_axis_name="s")


_EROWS = 64


def _emb_body(table_hbm, nodes_hbm, out_hbm, idx_v, rows_v, sem):
    cid = lax.axis_index("c")
    sid = lax.axis_index("s")
    wid = sid * 2 + cid
    per_w = (B * NP) // 32
    for j in range(per_w // _EROWS):
        base = wid * per_w + j * _EROWS
        pltpu.sync_copy(nodes_hbm.at[pl.ds(base, _EROWS)], idx_v)
        pltpu.async_copy(table_hbm.at[idx_v], rows_v, sem).wait()
        pltpu.sync_copy(rows_v, out_hbm.at[pl.ds(base, _EROWS)])


_emb_gather = pl.kernel(
    _emb_body,
    out_type=jax.ShapeDtypeStruct((B * NP, EMB), jnp.float32),
    mesh=_MESH,
    scratch_types=[
        pltpu.VMEM((_EROWS,), jnp.int32),
        pltpu.VMEM((_EROWS, EMB), jnp.float32),
        pltpu.SemaphoreType.DMA,
    ],
)


def _agg_body(x2_hbm, srcs_hbm, dsts_hbm, iota_hbm, iota2_hbm, zer_hbm,
              znp_hbm, params_hbm,
              acc_out, deg_out,
              src0, dst0, src1, dst1, rows0, rows1, deg_l, par_v,
              x_sh, acc_sh, gsem0, gsem1, ssem0, ssem1,
              isem0, isem1, dsem0, dsem1):
    cid = lax.axis_index("c")
    sid = lax.axis_index("s")

    pltpu.sync_copy(params_hbm, par_v)
    pv = par_v[pl.ds(0, 16)]
    idx2_base = pl.multiple_of(pv[0], 8)
    deg_flag = pv[1]

    pltpu.sync_copy(zer_hbm, rows1)
    for k in range(SUBBLK):
        base = sid * STRIPE + k * CHUNK
        pltpu.sync_copy(iota_hbm.at[pl.ds(base, CHUNK)], dst0)
        pltpu.sync_copy(
            iota2_hbm.at[pl.ds(idx2_base + cid * NP + base, CHUNK)], src0)
        pltpu.async_copy(x2_hbm.at[src0], rows0, gsem0).wait()
        pltpu.sync_copy(rows0, x_sh.at[dst0])
        pltpu.sync_copy(rows1, acc_sh.at[dst0])

    @pl.when(deg_flag == 1)
    def _():
        pltpu.sync_copy(znp_hbm, deg_l)

    plsc.subcore_barrier()

    edgebase = (cid * NCHUNK + sid * CPT) * CHUNK
    bufs = ((src0, dst0, rows0, gsem0, ssem0, isem0, dsem0),
            (src1, dst1, rows1, gsem1, ssem1, isem1, dsem1))

    def stage_src(c, src_v, isem):
        pltpu.async_copy(srcs_hbm.at[pl.ds(edgebase + c * CHUNK, CHUNK)],
                         src_v, isem)

    def stage_dst(c, dst_v, dsem):
        pltpu.async_copy(dsts_hbm.at[pl.ds(edgebase + c * CHUNK, CHUNK)],
                         dst_v, dsem)

    for parity in range(2):
        src_v, dst_v, rows_v, gsem, ssem, isem, dsem = bufs[parity]
        stage_src(parity, src_v, isem)
        stage_dst(parity, dst_v, dsem)
        pltpu.make_async_copy(srcs_hbm.at[pl.ds(0, CHUNK)], src_v, isem).wait()
        pltpu.async_copy(x_sh.at[src_v], rows_v, gsem)

    def pair(p, _):
        ones16 = jnp.ones((16,), jnp.float32)
        last = (CPT // 2) - 1
        for parity in range(2):
            src_v, dst_v, rows_v, gsem, ssem, isem, dsem = bufs[parity]
            c = 2 * p + parity
            pltpu.make_async_copy(x_sh.at[src_v], rows_v, gsem).wait()

            @pl.when(p < last)
            def _():
                stage_src(c + 2, src_v, isem)

            pltpu.make_async_copy(
                dsts_hbm.at[pl.ds(0, CHUNK)], dst_v, dsem).wait()
            pltpu.async_copy(rows_v, acc_sh.at[dst_v], ssem, add=True)

            @pl.when(deg_flag == 1)
            def _():
                for i in range(CHUNK // 16):
                    idx = dst_v[pl.ds(i * 16, 16)]
                    plsc.addupdate_scatter(deg_l, [idx], ones16)

            pltpu.make_async_copy(rows_v, acc_sh.at[dst_v], ssem).wait()

            @pl.when(p < last)
            def _():
                stage_dst(c + 2, dst_v, dsem)
                pltpu.make_async_copy(
                    srcs_hbm.at[pl.ds(0, CHUNK)], src_v, isem).wait()
                pltpu.async_copy(x_sh.at[src_v], rows_v, gsem)
        return 0

    lax.fori_loop(0, CPT // 2, pair, 0)
    plsc.subcore_barrier()

    for k in range(SUBBLK):
        base = sid * STRIPE + k * CHUNK
        pltpu.sync_copy(iota_hbm.at[pl.ds(base, CHUNK)], src0)
        pltpu.async_copy(acc_sh.at[src0], rows0, gsem0).wait()
        pltpu.sync_copy(rows0, acc_out.at[pl.ds(cid * NP + base, CHUNK)])

    @pl.when(deg_flag == 1)
    def _():
        pltpu.sync_copy(deg_l, deg_out.at[pl.ds((cid * TILES + sid) * NP, NP)])


_agg = pl.kernel(
    _agg_body,
    out_type=(jax.ShapeDtypeStruct((B * NP, HEMB), jnp.float32),
              jax.ShapeDtypeStruct((B * TILES * NP,), jnp.float32)),
    mesh=_MESH,
    compiler_params=pltpu.CompilerParams(needs_layout_passes=False, use_tc_tiling_on_sc=False),
    scratch_types=[
        pltpu.VMEM((CHUNK,), jnp.int32),
        pltpu.VMEM((CHUNK,), jnp.int32),
        pltpu.VMEM((CHUNK,), jnp.int32),
        pltpu.VMEM((CHUNK,), jnp.int32),
        pltpu.VMEM((CHUNK, HEMB), jnp.float32),
        pltpu.VMEM((CHUNK, HEMB), jnp.float32),
        pltpu.VMEM((NP,), jnp.float32),
        pltpu.VMEM((16,), jnp.int32),
        pltpu.VMEM_SHARED((NP, HEMB), jnp.float32),
        pltpu.VMEM_SHARED((NP, HEMB), jnp.float32),
        pltpu.SemaphoreType.DMA,
        pltpu.SemaphoreType.DMA,
        pltpu.SemaphoreType.DMA,
        pltpu.SemaphoreType.DMA,
        pltpu.SemaphoreType.DMA,
        pltpu.SemaphoreType.DMA,
        pltpu.SemaphoreType.DMA,
        pltpu.SemaphoreType.DMA,
    ],
)


_MM_BLK = 1024


def _mm_body(relu, acc0_ref, acc1_ref, deg_ref, x_ref, w_ref, r_ref, b_ref,
             out_ref):
    d = jnp.sum(deg_ref[...], axis=0)[:, None]
    inv = 1.0 / jnp.maximum(d, 1.0)
    a = jnp.concatenate([acc0_ref[...], acc1_ref[...]], axis=1) * inv
    h = (jnp.dot(a, w_ref[...], preferred_element_type=jnp.float32)
         + jnp.dot(x_ref[...], r_ref[...], preferred_element_type=jnp.float32)
         + b_ref[...])
    if relu:
        h = jnp.maximum(h, 0.0)
    rows = pl.program_id(0) * _MM_BLK + lax.broadcasted_iota(
        jnp.int32, (_MM_BLK, 1), 0)
    h = jnp.where((rows % NP) < N, h, 0.0)
    out_ref[...] = h


def _make_mm(relu):
    return pl.pallas_call(
        functools.partial(_mm_body, relu),
        grid=((B * NP) // _MM_BLK,),
        in_specs=[
            pl.BlockSpec((_MM_BLK, HEMB), lambda i: (i, 0)),
            pl.BlockSpec((_MM_BLK, HEMB), lambda i: (i, 0)),
            pl.BlockSpec((TILES, _MM_BLK),
                         lambda i: (i // (NP // _MM_BLK), i % (NP // _MM_BLK))),
            pl.BlockSpec((_MM_BLK, EMB), lambda i: (i, 0)),
            pl.BlockSpec((EMB, EMB), lambda i: (0, 0)),
            pl.BlockSpec((EMB, EMB), lambda i: (0, 0)),
            pl.BlockSpec((1, EMB), lambda i: (0, 0)),
        ],
        out_specs=pl.BlockSpec((_MM_BLK, EMB), lambda i: (i, 0)),
        out_shape=jax.ShapeDtypeStruct((B * NP, EMB), jnp.float32),
    )


_mm_relu = _make_mm(True)
_mm_lin = _make_mm(False)


def kernel(nodes, edges, types, node_table, W0, R0, b0, W1, R1, b1):
    del types
    f32 = jnp.float32

    table_pad = jnp.concatenate(
        [node_table.astype(f32), jnp.zeros((8, EMB), f32)], axis=0)
    zero_row = jnp.int32(node_table.shape[0])

    nodes_pad = jnp.concatenate(
        [nodes.astype(jnp.int32),
         jnp.full((B, NP - N), zero_row, jnp.int32)], axis=1)
    nodes_flat = nodes_pad.reshape(B * NP)

    src = edges[:, 0, :].astype(jnp.int32)
    dst = edges[:, 1, :].astype(jnp.int32)
    pad = jnp.full((B, EP - E), N, jnp.int32)
    srcs_rs = jnp.concatenate([src, pad], axis=1).reshape(B * EP)
    dsts_rs = jnp.concatenate([dst, pad], axis=1).reshape(B * EP)

    zer = jnp.zeros((CHUNK, HEMB), f32)
    znp = jnp.zeros((NP,), f32)
    iota_np = jnp.arange(NP, dtype=jnp.int32)
    r = jnp.arange(B * NP, dtype=jnp.int32)
    iota2 = jnp.concatenate([2 * r, 2 * r + 1])
    par0 = jnp.zeros((16,), jnp.int32).at[1].set(1)
    par1 = jnp.zeros((16,), jnp.int32).at[0].set(B * NP)

    x0 = _emb_gather(table_pad, nodes_flat)

    def layer(x, W, R, b, mm):
        x2 = x.reshape(2 * B * NP, HEMB)
        acc_h0, deg = _agg(x2, srcs_rs, dsts_rs, iota_np, iota2, zer, znp, par0)
        dep = (deg[:16] < -1e30).astype(jnp.int32)
        acc_h1, _ = _agg(x2, srcs_rs, dsts_rs, iota_np, iota2, zer, znp,
                         par1 + dep)
        return mm(acc_h0, acc_h1, deg.reshape(B * TILES, NP),
                  x, W, R, b.reshape(1, EMB))

    x1 = layer(x0, W0, R0, b0, _mm_relu)
    x2 = layer(x1, W1, R1, b1, _mm_lin)
    return x2.reshape(B, NP, EMB)[:, :N, :]
